# Initial kernel scaffold; baseline (speedup 1.0000x reference)
import jax
import jax.numpy as jnp
from jax.experimental import pallas as pl


def kernel(AtomBondGraph_edges, BondAngleGraph_edges, AngleDihedralGraph_edges, x, bond_attr, bond_lengths, bond_angles, dihedral_angles, atom_batch, num_graphs, masked_atom_indices, masked_bond_indices, masked_angle_indices, masked_dihedral_indices, params):
    raise NotImplementedError("write your pallas kernel here")


# speedup vs baseline: 1.6231x; 1.6231x over previous
import functools
import math

import jax
import jax.numpy as jnp
import numpy as np
from jax.experimental import pallas as pl
from jax.experimental.pallas import tpu as pltpu

LATENT = 128
GAMMA = 10.0
ATOM_DIMS = (123, 16, 16, 9, 14, 11)
BOND_DIMS = (12, 27, 7)
BOND_CENTERS = np.arange(0.0, 2.0, 0.1).astype(np.float32)
ANGLE_CENTERS = np.arange(0.0, np.pi, 0.1).astype(np.float32)
DIHED_CENTERS = np.arange(-np.pi, np.pi, 0.2).astype(np.float32)


def _offsets(dims):
    offs = [0]
    for d in dims[:-1]:
        offs.append(offs[-1] + d)
    return tuple(offs)


def _embed_body(x_ref, t_ref, o_ref, *, offs, k_pad):
    x = x_ref[...]
    b = x.shape[0]
    iota = jax.lax.broadcasted_iota(jnp.int32, (b, k_pad), 1)
    oh = jnp.zeros((b, k_pad), jnp.float32)
    for f, off in enumerate(offs):
        oh = oh + (iota == (x[:, f][:, None] + off)).astype(jnp.float32)
    o_ref[...] = jnp.dot(oh, t_ref[...], preferred_element_type=jnp.float32)


def _embed_call(x, tables, dims, block):
    n = x.shape[0]
    offs = _offsets(dims)
    k = sum(dims)
    k_pad = ((k + 7) // 8) * 8
    t = jnp.zeros((k_pad, LATENT), jnp.float32)
    row = 0
    for tab, d in zip(tables, dims):
        t = jax.lax.dynamic_update_slice(t, tab, (row, 0))
        row += d
    grid = n // block
    return pl.pallas_call(
        functools.partial(_embed_body, offs=offs, k_pad=k_pad),
        grid=(grid,),
        in_specs=[
            pl.BlockSpec((block, len(dims)), lambda i: (i, 0)),
            pl.BlockSpec((k_pad, LATENT), lambda i: (0, 0)),
        ],
        out_specs=pl.BlockSpec((block, LATENT), lambda i: (i, 0)),
        out_shape=jax.ShapeDtypeStruct((n, LATENT), jnp.float32),
    )(x, t)


def _bondfeat_body(x_ref, v_ref, t_ref, b_ref, c_ref, o_ref, *, offs, k_pad):
    x = x_ref[...]
    v = v_ref[0, 0, :]
    b = x.shape[0]
    iota = jax.lax.broadcasted_iota(jnp.int32, (b, k_pad), 1)
    feat = jnp.zeros((b, k_pad), jnp.float32)
    for f, off in enumerate(offs):
        feat = feat + (iota == (x[:, f][:, None] + off)).astype(jnp.float32)
    cvec = c_ref[...]
    feat = feat + jnp.exp(-GAMMA * (v[:, None] - cvec[None, :]) ** 2)
    o_ref[...] = (
        jnp.dot(feat, t_ref[...], preferred_element_type=jnp.float32)
        + b_ref[...][None, :]
    )


def _bondfeat_call(bond_attr, bond_lengths, tables, rbf_p, block):
    n = bond_attr.shape[0]
    offs = _offsets(BOND_DIMS)
    k_emb = sum(BOND_DIMS)
    n_c = BOND_CENTERS.shape[0]
    k = k_emb + n_c
    k_pad = ((k + 7) // 8) * 8
    t = jnp.zeros((k_pad, LATENT), jnp.float32)
    row = 0
    for tab, d in zip(tables, BOND_DIMS):
        t = jax.lax.dynamic_update_slice(t, tab, (row, 0))
        row += d
    t = jax.lax.dynamic_update_slice(t, rbf_p["W"], (k_emb, 0))
    c_full = np.full((k_pad,), 1.0e4, np.float32)
    c_full[k_emb:k_emb + n_c] = BOND_CENTERS
    grid = n // block
    return pl.pallas_call(
        functools.partial(_bondfeat_body, offs=offs, k_pad=k_pad),
        grid=(grid,),
        in_specs=[
            pl.BlockSpec((block, 3), lambda i: (i, 0)),
            pl.BlockSpec((1, 1, block), lambda i: (i, 0, 0)),
            pl.BlockSpec((k_pad, LATENT), lambda i: (0, 0)),
            pl.BlockSpec((LATENT,), lambda i: (0,)),
            pl.BlockSpec((k_pad,), lambda i: (0,)),
        ],
        out_specs=pl.BlockSpec((block, LATENT), lambda i: (i, 0)),
        out_shape=jax.ShapeDtypeStruct((n, LATENT), jnp.float32),
    )(bond_attr, bond_lengths.reshape(grid, 1, block), t, rbf_p["b"],
      jnp.asarray(c_full))


def _rbf_body(v_ref, w_ref, b_ref, c_ref, o_ref):
    v = v_ref[0, 0, :]
    cvec = c_ref[...]
    r = jnp.exp(-GAMMA * (v[:, None] - cvec[None, :]) ** 2)
    o_ref[...] = (
        jnp.dot(r, w_ref[...], preferred_element_type=jnp.float32)
        + b_ref[...][None, :]
    )


def _rbf_call(v, centers, p, block):
    n = v.shape[0]
    c = centers.shape[0]
    grid = n // block
    return pl.pallas_call(
        _rbf_body,
        grid=(grid,),
        in_specs=[
            pl.BlockSpec((1, 1, block), lambda i: (i, 0, 0)),
            pl.BlockSpec((c, LATENT), lambda i: (0, 0)),
            pl.BlockSpec((LATENT,), lambda i: (0,)),
            pl.BlockSpec((c,), lambda i: (0,)),
        ],
        out_specs=pl.BlockSpec((block, LATENT), lambda i: (i, 0)),
        out_shape=jax.ShapeDtypeStruct((n, LATENT), jnp.float32),
    )(v.reshape(grid, 1, block), p["W"], p["b"], jnp.asarray(centers))


def _mlp_body(agg_ref, skip_ref, w1_ref, b1_ref, w2_ref, b2_ref, g_ref,
              bl_ref, o_ref, *, last_act):
    agg = agg_ref[...]
    h = jnp.dot(agg, w1_ref[...], preferred_element_type=jnp.float32)
    h = jnp.maximum(h + b1_ref[...][None, :], 0.0)
    o = jnp.dot(h, w2_ref[...], preferred_element_type=jnp.float32)
    o = o + b2_ref[...][None, :]
    mu = jnp.mean(o, axis=-1, keepdims=True)
    var = jnp.mean((o - mu) ** 2, axis=-1, keepdims=True)
    o = (o - mu) * jax.lax.rsqrt(var + 1e-5)
    o = o * g_ref[...][None, :] + bl_ref[...][None, :]
    if last_act:
        o = jnp.maximum(o, 0.0)
    o_ref[...] = o + skip_ref[...]


def _mlp_call(agg, skip, p, last_act, block):
    n = agg.shape[0]
    grid = n // block
    return pl.pallas_call(
        functools.partial(_mlp_body, last_act=last_act),
        grid=(grid,),
        in_specs=[
            pl.BlockSpec((block, LATENT), lambda i: (i, 0)),
            pl.BlockSpec((block, LATENT), lambda i: (i, 0)),
            pl.BlockSpec((LATENT, 2 * LATENT), lambda i: (0, 0)),
            pl.BlockSpec((2 * LATENT,), lambda i: (0,)),
            pl.BlockSpec((2 * LATENT, LATENT), lambda i: (0, 0)),
            pl.BlockSpec((LATENT,), lambda i: (0,)),
            pl.BlockSpec((LATENT,), lambda i: (0,)),
            pl.BlockSpec((LATENT,), lambda i: (0,)),
        ],
        out_specs=pl.BlockSpec((block, LATENT), lambda i: (i, 0)),
        out_shape=jax.ShapeDtypeStruct((n, LATENT), jnp.float32),
    )(agg, skip, p["W1"], p["b1"], p["W2"], p["b2"], p["ln_g"], p["ln_b"])


def _pool_body(batch_ref, node_ref, o_ref, seg_ref, cnt_ref, *, n_graphs,
               n_steps):
    i = pl.program_id(0)

    @pl.when(i == 0)
    def _():
        seg_ref[...] = jnp.zeros_like(seg_ref)
        cnt_ref[...] = jnp.zeros_like(cnt_ref)

    batch = batch_ref[0, 0, :]
    b = batch.shape[0]
    gid = jax.lax.broadcasted_iota(jnp.int32, (n_graphs, b), 0)
    oh = (gid == batch[None, :]).astype(jnp.float32)
    seg_ref[...] += jnp.dot(oh, node_ref[...],
                            preferred_element_type=jnp.float32)
    cnt_ref[...] += jnp.dot(oh, jnp.ones((b, LATENT), jnp.float32),
                            preferred_element_type=jnp.float32)

    @pl.when(i == n_steps - 1)
    def _():
        o_ref[...] = seg_ref[...] / jnp.maximum(cnt_ref[...], 1.0)


def _pool_call(node, batch, n_graphs, block):
    n = node.shape[0]
    grid = n // block
    return pl.pallas_call(
        functools.partial(_pool_body, n_graphs=n_graphs, n_steps=grid),
        grid=(grid,),
        in_specs=[
            pl.BlockSpec((1, 1, block), lambda i: (i, 0, 0)),
            pl.BlockSpec((block, LATENT), lambda i: (i, 0)),
        ],
        out_specs=pl.BlockSpec((n_graphs, LATENT), lambda i: (0, 0)),
        out_shape=jax.ShapeDtypeStruct((n_graphs, LATENT), jnp.float32),
        scratch_shapes=[
            pltpu.VMEM((n_graphs, LATENT), jnp.float32),
            pltpu.VMEM((n_graphs, LATENT), jnp.float32),
        ],
    )(batch.reshape(grid, 1, block), node)


def _message_agg(node_hidden, edge_hidden, edges, n_nodes):
    src, dst = edges[0], edges[1]
    msg = jnp.maximum(jnp.take(node_hidden, src, axis=0) + edge_hidden, 0.0)
    return jax.ops.segment_sum(msg, dst, num_segments=n_nodes)


def _block(node_hidden, edge_hidden, edges, p, last_act, block):
    agg = _message_agg(node_hidden, edge_hidden, edges, node_hidden.shape[0])
    return _mlp_call(agg, node_hidden, p, last_act, block)


def kernel(AtomBondGraph_edges, BondAngleGraph_edges, AngleDihedralGraph_edges,
           x, bond_attr, bond_lengths, bond_angles, dihedral_angles,
           atom_batch, num_graphs, masked_atom_indices, masked_bond_indices,
           masked_angle_indices, masked_dihedral_indices, params):
    n_atoms = x.shape[0]
    n_bonds = bond_attr.shape[0]

    for i, d in enumerate(ATOM_DIMS):
        x = x.at[masked_atom_indices, i].set(d - 1)
    for i, d in enumerate(BOND_DIMS):
        bond_attr = bond_attr.at[masked_bond_indices, i].set(d - 1)
    bond_lengths = bond_lengths.at[masked_bond_indices].set(0.0)
    bond_angles = bond_angles.at[masked_angle_indices].set(0.0)
    dihedral_angles = dihedral_angles.at[masked_dihedral_indices].set(0.0)

    ab = 2000
    eb = 4000

    node_hidden = _embed_call(x, params["init_atom_emb"], ATOM_DIMS, ab)
    bond_hidden = _bondfeat_call(bond_attr, bond_lengths,
                                 params["init_bond_emb"],
                                 params["init_bond_rbf"], eb)
    angle_hidden = _rbf_call(bond_angles, ANGLE_CENTERS,
                             params["init_angle_rbf"], eb)

    cur_dihedral_hidden = None
    n_layers = len(params["layers"])
    for l in range(n_layers):
        lp = params["layers"][l]
        last_act = (l != n_layers - 1)
        new_node = _block(node_hidden, bond_hidden, AtomBondGraph_edges,
                          lp["ab_block"], last_act, ab)
        cur_edge_hidden = _bondfeat_call(bond_attr, bond_lengths,
                                         lp["bond_emb"], lp["bond_rbf"], eb)
        new_bond = _block(cur_edge_hidden, angle_hidden, BondAngleGraph_edges,
                          lp["ba_block"], last_act, eb)
        cur_angle_hidden = _rbf_call(bond_angles, ANGLE_CENTERS,
                                     lp["angle_rbf"], eb)
        cur_dihedral_hidden = _rbf_call(dihedral_angles, DIHED_CENTERS,
                                        lp["dihed_rbf"], eb)
        new_angle = _block(cur_angle_hidden, cur_dihedral_hidden,
                           AngleDihedralGraph_edges, lp["ad_block"],
                           last_act, eb)
        node_hidden, bond_hidden, angle_hidden = new_node, new_bond, new_angle

    graph_repr = _pool_call(node_hidden, atom_batch, 128, 1000)
    return (node_hidden, bond_hidden, angle_hidden, cur_dihedral_hidden,
            graph_repr)


# speedup vs baseline: 1.9414x; 1.1961x over previous
import functools
import math

import jax
import jax.numpy as jnp
import numpy as np
from jax import lax
from jax.experimental import pallas as pl
from jax.experimental.pallas import tpu as pltpu
from jax.experimental.pallas import tpu_sc as plsc

LATENT = 128
GAMMA = 10.0
ATOM_DIMS = (123, 16, 16, 9, 14, 11)
BOND_DIMS = (12, 27, 7)
BOND_CENTERS = np.arange(0.0, 2.0, 0.1).astype(np.float32)
ANGLE_CENTERS = np.arange(0.0, np.pi, 0.1).astype(np.float32)
DIHED_CENTERS = np.arange(-np.pi, np.pi, 0.2).astype(np.float32)


def _offsets(dims):
    offs = [0]
    for d in dims[:-1]:
        offs.append(offs[-1] + d)
    return tuple(offs)


def _embed_body(x_ref, t_ref, o_ref, *, offs, k_pad):
    x = x_ref[...]
    b = x.shape[0]
    iota = jax.lax.broadcasted_iota(jnp.int32, (b, k_pad), 1)
    oh = jnp.zeros((b, k_pad), jnp.float32)
    for f, off in enumerate(offs):
        oh = oh + (iota == (x[:, f][:, None] + off)).astype(jnp.float32)
    o_ref[...] = jnp.dot(oh, t_ref[...], preferred_element_type=jnp.float32)


def _embed_call(x, tables, dims, block):
    n = x.shape[0]
    offs = _offsets(dims)
    k = sum(dims)
    k_pad = ((k + 7) // 8) * 8
    t = jnp.zeros((k_pad, LATENT), jnp.float32)
    row = 0
    for tab, d in zip(tables, dims):
        t = jax.lax.dynamic_update_slice(t, tab, (row, 0))
        row += d
    grid = n // block
    return pl.pallas_call(
        functools.partial(_embed_body, offs=offs, k_pad=k_pad),
        grid=(grid,),
        in_specs=[
            pl.BlockSpec((block, len(dims)), lambda i: (i, 0)),
            pl.BlockSpec((k_pad, LATENT), lambda i: (0, 0)),
        ],
        out_specs=pl.BlockSpec((block, LATENT), lambda i: (i, 0)),
        out_shape=jax.ShapeDtypeStruct((n, LATENT), jnp.float32),
    )(x, t)


def _bondfeat_body(x_ref, v_ref, t_ref, b_ref, c_ref, o_ref, *, offs, k_pad):
    x = x_ref[...]
    v = v_ref[0, 0, :]
    b = x.shape[0]
    iota = jax.lax.broadcasted_iota(jnp.int32, (b, k_pad), 1)
    feat = jnp.zeros((b, k_pad), jnp.float32)
    for f, off in enumerate(offs):
        feat = feat + (iota == (x[:, f][:, None] + off)).astype(jnp.float32)
    cvec = c_ref[...]
    feat = feat + jnp.exp(-GAMMA * (v[:, None] - cvec[None, :]) ** 2)
    o_ref[...] = (
        jnp.dot(feat, t_ref[...], preferred_element_type=jnp.float32)
        + b_ref[...][None, :]
    )


def _bondfeat_call(bond_attr, bond_lengths, tables, rbf_p, block):
    n = bond_attr.shape[0]
    offs = _offsets(BOND_DIMS)
    k_emb = sum(BOND_DIMS)
    n_c = BOND_CENTERS.shape[0]
    k = k_emb + n_c
    k_pad = ((k + 7) // 8) * 8
    t = jnp.zeros((k_pad, LATENT), jnp.float32)
    row = 0
    for tab, d in zip(tables, BOND_DIMS):
        t = jax.lax.dynamic_update_slice(t, tab, (row, 0))
        row += d
    t = jax.lax.dynamic_update_slice(t, rbf_p["W"], (k_emb, 0))
    c_full = np.full((k_pad,), 1.0e4, np.float32)
    c_full[k_emb:k_emb + n_c] = BOND_CENTERS
    grid = n // block
    return pl.pallas_call(
        functools.partial(_bondfeat_body, offs=offs, k_pad=k_pad),
        grid=(grid,),
        in_specs=[
            pl.BlockSpec((block, 3), lambda i: (i, 0)),
            pl.BlockSpec((1, 1, block), lambda i: (i, 0, 0)),
            pl.BlockSpec((k_pad, LATENT), lambda i: (0, 0)),
            pl.BlockSpec((LATENT,), lambda i: (0,)),
            pl.BlockSpec((k_pad,), lambda i: (0,)),
        ],
        out_specs=pl.BlockSpec((block, LATENT), lambda i: (i, 0)),
        out_shape=jax.ShapeDtypeStruct((n, LATENT), jnp.float32),
    )(bond_attr, bond_lengths.reshape(grid, 1, block), t, rbf_p["b"],
      jnp.asarray(c_full))


def _rbf_body(v_ref, w_ref, b_ref, c_ref, o_ref):
    v = v_ref[0, 0, :]
    cvec = c_ref[...]
    r = jnp.exp(-GAMMA * (v[:, None] - cvec[None, :]) ** 2)
    o_ref[...] = (
        jnp.dot(r, w_ref[...], preferred_element_type=jnp.float32)
        + b_ref[...][None, :]
    )


def _rbf_call(v, centers, p, block):
    n = v.shape[0]
    c = centers.shape[0]
    grid = n // block
    return pl.pallas_call(
        _rbf_body,
        grid=(grid,),
        in_specs=[
            pl.BlockSpec((1, 1, block), lambda i: (i, 0, 0)),
            pl.BlockSpec((c, LATENT), lambda i: (0, 0)),
            pl.BlockSpec((LATENT,), lambda i: (0,)),
            pl.BlockSpec((c,), lambda i: (0,)),
        ],
        out_specs=pl.BlockSpec((block, LATENT), lambda i: (i, 0)),
        out_shape=jax.ShapeDtypeStruct((n, LATENT), jnp.float32),
    )(v.reshape(grid, 1, block), p["W"], p["b"], jnp.asarray(centers))


def _mlp_body(agg_ref, skip_ref, w1_ref, b1_ref, w2_ref, b2_ref, g_ref,
              bl_ref, o_ref, *, last_act):
    agg = agg_ref[...]
    h = jnp.dot(agg, w1_ref[...], preferred_element_type=jnp.float32)
    h = jnp.maximum(h + b1_ref[...][None, :], 0.0)
    o = jnp.dot(h, w2_ref[...], preferred_element_type=jnp.float32)
    o = o + b2_ref[...][None, :]
    mu = jnp.mean(o, axis=-1, keepdims=True)
    var = jnp.mean((o - mu) ** 2, axis=-1, keepdims=True)
    o = (o - mu) * jax.lax.rsqrt(var + 1e-5)
    o = o * g_ref[...][None, :] + bl_ref[...][None, :]
    if last_act:
        o = jnp.maximum(o, 0.0)
    o_ref[...] = o + skip_ref[...]


def _mlp_call(agg, skip, p, last_act, block):
    n = skip.shape[0]
    grid = n // block
    return pl.pallas_call(
        functools.partial(_mlp_body, last_act=last_act),
        grid=(grid,),
        in_specs=[
            pl.BlockSpec((block, LATENT), lambda i: (i, 0)),
            pl.BlockSpec((block, LATENT), lambda i: (i, 0)),
            pl.BlockSpec((LATENT, 2 * LATENT), lambda i: (0, 0)),
            pl.BlockSpec((2 * LATENT,), lambda i: (0,)),
            pl.BlockSpec((2 * LATENT, LATENT), lambda i: (0, 0)),
            pl.BlockSpec((LATENT,), lambda i: (0,)),
            pl.BlockSpec((LATENT,), lambda i: (0,)),
            pl.BlockSpec((LATENT,), lambda i: (0,)),
        ],
        out_specs=pl.BlockSpec((block, LATENT), lambda i: (i, 0)),
        out_shape=jax.ShapeDtypeStruct((n, LATENT), jnp.float32),
    )(agg, skip, p["W1"], p["b1"], p["W2"], p["b2"], p["ln_g"], p["ln_b"])


def _pool_body(batch_ref, node_ref, o_ref, seg_ref, cnt_ref, *, n_graphs,
               n_steps):
    i = pl.program_id(0)

    @pl.when(i == 0)
    def _():
        seg_ref[...] = jnp.zeros_like(seg_ref)
        cnt_ref[...] = jnp.zeros_like(cnt_ref)

    batch = batch_ref[0, 0, :]
    b = batch.shape[0]
    gid = jax.lax.broadcasted_iota(jnp.int32, (n_graphs, b), 0)
    oh = (gid == batch[None, :]).astype(jnp.float32)
    seg_ref[...] += jnp.dot(oh, node_ref[...],
                            preferred_element_type=jnp.float32)
    cnt_ref[...] += jnp.dot(oh, jnp.ones((b, LATENT), jnp.float32),
                            preferred_element_type=jnp.float32)

    @pl.when(i == n_steps - 1)
    def _():
        o_ref[...] = seg_ref[...] / jnp.maximum(cnt_ref[...], 1.0)


def _pool_call(node, batch, n_graphs, block):
    n = node.shape[0]
    grid = n // block
    return pl.pallas_call(
        functools.partial(_pool_body, n_graphs=n_graphs, n_steps=grid),
        grid=(grid,),
        in_specs=[
            pl.BlockSpec((1, 1, block), lambda i: (i, 0, 0)),
            pl.BlockSpec((block, LATENT), lambda i: (i, 0)),
        ],
        out_specs=pl.BlockSpec((n_graphs, LATENT), lambda i: (0, 0)),
        out_shape=jax.ShapeDtypeStruct((n_graphs, LATENT), jnp.float32),
        scratch_shapes=[
            pltpu.VMEM((n_graphs, LATENT), jnp.float32),
            pltpu.VMEM((n_graphs, LATENT), jnp.float32),
        ],
    )(batch.reshape(grid, 1, block), node)


_SC_R = 8192
_SC_LOGR = 13
_SC_G = 128
_SC_PAD = 16 * _SC_G + 8


def _sc_agg_body(node_hbm, edge_hbm, srcs_hbm, eids_hbm, dls_hbm, cks_hbm,
                 co_hbm, out_hbm, sidx, eidx, ckb, dlb, dl2d, rows_a, rows_b,
                 zbuf, cob, acc, sem_a, sem_b, *, n_chunks):
    cid = lax.axis_index("c")
    sid = lax.axis_index("s")
    r = _SC_R
    g = _SC_G
    pltpu.sync_copy(co_hbm, cob)
    cov1 = cob[pl.ds(0, 16)]
    cov2 = cob[pl.ds(16, 16)]

    def co_at(i):
        return cov1[i] if i < 16 else cov2[i - 16]

    def zero_z(i, _):
        for q in range(8):
            zbuf[i, pl.ds(16 * q, 16)] = jnp.zeros((16,), jnp.float32)
        return _

    lax.fori_loop(0, g, zero_z, None)
    row0 = sid * (r // 16)

    for ci in range(n_chunks):
        lo = ci * r
        start = pl.multiple_of(co_at(ci) & ~7, 8)
        end = co_at(ci + 1)
        n_g = ((end - start) + (16 * g - 1)) >> 11

        @pl.when(cid == ci % 2)
        def _zero():
            for o in range(0, r // 16, g):
                pltpu.sync_copy(zbuf, acc.at[pl.ds(row0 + o, g)])

        plsc.subcore_barrier()

        def group(gi, _, start=start, ci=ci):
            e0 = pl.multiple_of(start + (gi * 16 + sid) * _SC_G, 8)
            pltpu.sync_copy(srcs_hbm.at[pl.ds(e0, _SC_G)], sidx)
            pltpu.sync_copy(eids_hbm.at[pl.ds(e0, _SC_G)], eidx)
            pltpu.sync_copy(cks_hbm.at[pl.ds(e0, _SC_G)], ckb)
            pltpu.sync_copy(dls_hbm.at[pl.ds(e0, _SC_G)], dlb)
            cp_a = pltpu.async_copy(node_hbm.at[sidx], rows_a, sem_a)
            cp_b = pltpu.async_copy(edge_hbm.at[eidx], rows_b, sem_b)
            cp_a.wait()
            cp_b.wait()

            def relu_row(rr, _):
                for q in range(8):
                    sl = pl.ds(16 * q, 16)
                    rows_a[rr, sl] = jnp.maximum(
                        rows_a[rr, sl] + rows_b[rr, sl], 0.0)
                return _

            lax.fori_loop(0, _SC_G, relu_row, None)
            for q in range(8):
                sl = pl.ds(16 * q, 16)
                dl2d[0, sl] = jnp.where(ckb[sl] == ci, dlb[sl], _SC_R)
            pltpu.sync_copy(rows_a, acc.at[dl2d.at[0]], add=True)
            return _

        @pl.when(cid == ci % 2)
        def _scan(n_g=n_g, group=group):
            lax.fori_loop(0, n_g, group, None)

        plsc.subcore_barrier()

        @pl.when(cid == ci % 2)
        def _flush(lo=lo):
            out_base = lo + row0
            for o in range(0, r // 16, g):
                pltpu.sync_copy(acc.at[pl.ds(row0 + o, g)],
                                out_hbm.at[pl.ds(out_base + o, g)])

        plsc.subcore_barrier()


def _sc_prep(edges, n_nodes):
    src = edges[0].astype(jnp.int32)
    dst = edges[1].astype(jnp.int32)
    n_chunks = -(-n_nodes // _SC_R)
    if n_chunks % 2:
        n_chunks += 1
    perm = jnp.argsort(dst).astype(jnp.int32)
    dst_s = jnp.take(dst, perm)
    src_s = jnp.take(src, perm)
    dl_s = dst_s & (_SC_R - 1)
    ck_s = dst_s >> _SC_LOGR
    co = jnp.searchsorted(
        dst_s, jnp.arange(n_chunks + 1, dtype=jnp.int32) * _SC_R
    ).astype(jnp.int32)
    cob = jnp.full((32,), src.shape[0], jnp.int32)
    cob = jax.lax.dynamic_update_slice(cob, co, (0,))
    zpad = jnp.zeros((_SC_PAD,), jnp.int32)
    return dict(
        n_chunks=n_chunks,
        src_s=jnp.concatenate([src_s, zpad]),
        eid_s=jnp.concatenate([perm, zpad]),
        dl_s=jnp.concatenate([dl_s, zpad]),
        ck_s=jnp.concatenate([ck_s, zpad + (1 << 30)]),
        co=cob,
    )


def _sc_message_agg(node_hidden, edge_hidden, prep, n_nodes):
    n_chunks = prep["n_chunks"]
    n_pad = n_chunks * _SC_R
    kfn = functools.partial(
        pl.kernel,
        mesh=plsc.VectorSubcoreMesh(core_axis_name="c", subcore_axis_name="s"),
        out_type=jax.ShapeDtypeStruct((n_pad, LATENT), jnp.float32),
        scratch_types=[
            pltpu.VMEM((_SC_G,), jnp.int32),
            pltpu.VMEM((_SC_G,), jnp.int32),
            pltpu.VMEM((_SC_G,), jnp.int32),
            pltpu.VMEM((_SC_G,), jnp.int32),
            pltpu.VMEM((1, 128), jnp.int32),
            pltpu.VMEM((_SC_G, LATENT), jnp.float32),
            pltpu.VMEM((_SC_G, LATENT), jnp.float32),
            pltpu.VMEM((_SC_G, LATENT), jnp.float32),
            pltpu.VMEM((32,), jnp.int32),
            pltpu.VMEM_SHARED((_SC_R + 8, LATENT), jnp.float32),
            pltpu.SemaphoreType.DMA,
            pltpu.SemaphoreType.DMA,
        ],
    )(functools.partial(_sc_agg_body, n_chunks=n_chunks))
    return kfn(node_hidden, edge_hidden, prep["src_s"], prep["eid_s"],
               prep["dl_s"], prep["ck_s"], prep["co"])


def _message_agg(node_hidden, edge_hidden, prep, n_nodes):
    return _sc_message_agg(node_hidden, edge_hidden, prep, n_nodes)


def _block(node_hidden, edge_hidden, prep, p, last_act, block):
    agg = _message_agg(node_hidden, edge_hidden, prep,
                       node_hidden.shape[0])
    return _mlp_call(agg, node_hidden, p, last_act, block)


def kernel(AtomBondGraph_edges, BondAngleGraph_edges, AngleDihedralGraph_edges,
           x, bond_attr, bond_lengths, bond_angles, dihedral_angles,
           atom_batch, num_graphs, masked_atom_indices, masked_bond_indices,
           masked_angle_indices, masked_dihedral_indices, params):
    n_atoms = x.shape[0]
    n_bonds = bond_attr.shape[0]

    for i, d in enumerate(ATOM_DIMS):
        x = x.at[masked_atom_indices, i].set(d - 1)
    for i, d in enumerate(BOND_DIMS):
        bond_attr = bond_attr.at[masked_bond_indices, i].set(d - 1)
    bond_lengths = bond_lengths.at[masked_bond_indices].set(0.0)
    bond_angles = bond_angles.at[masked_angle_indices].set(0.0)
    dihedral_angles = dihedral_angles.at[masked_dihedral_indices].set(0.0)

    ab = 2000
    eb = 4000

    node_hidden = _embed_call(x, params["init_atom_emb"], ATOM_DIMS, ab)
    bond_hidden = _bondfeat_call(bond_attr, bond_lengths,
                                 params["init_bond_emb"],
                                 params["init_bond_rbf"], eb)
    angle_hidden = _rbf_call(bond_angles, ANGLE_CENTERS,
                             params["init_angle_rbf"], eb)

    ab_prep = _sc_prep(AtomBondGraph_edges, n_atoms)
    ba_prep = _sc_prep(BondAngleGraph_edges, n_bonds)
    ad_prep = _sc_prep(AngleDihedralGraph_edges, bond_angles.shape[0])

    cur_dihedral_hidden = None
    n_layers = len(params["layers"])
    for l in range(n_layers):
        lp = params["layers"][l]
        last_act = (l != n_layers - 1)
        new_node = _block(node_hidden, bond_hidden, ab_prep,
                          lp["ab_block"], last_act, ab)
        cur_edge_hidden = _bondfeat_call(bond_attr, bond_lengths,
                                         lp["bond_emb"], lp["bond_rbf"], eb)
        new_bond = _block(cur_edge_hidden, angle_hidden, ba_prep,
                          lp["ba_block"], last_act, eb)
        cur_angle_hidden = _rbf_call(bond_angles, ANGLE_CENTERS,
                                     lp["angle_rbf"], eb)
        cur_dihedral_hidden = _rbf_call(dihedral_angles, DIHED_CENTERS,
                                        lp["dihed_rbf"], eb)
        new_angle = _block(cur_angle_hidden, cur_dihedral_hidden,
                           ad_prep, lp["ad_block"],
                           last_act, eb)
        node_hidden, bond_hidden, angle_hidden = new_node, new_bond, new_angle

    graph_repr = _pool_call(node_hidden, atom_batch, 128, 1000)
    return (node_hidden, bond_hidden, angle_hidden, cur_dihedral_hidden,
            graph_repr)


# speedup vs baseline: 1.9513x; 1.0051x over previous
import functools
import math

import jax
import jax.numpy as jnp
import numpy as np
from jax import lax
from jax.experimental import pallas as pl
from jax.experimental.pallas import tpu as pltpu
from jax.experimental.pallas import tpu_sc as plsc

LATENT = 128
GAMMA = 10.0
ATOM_DIMS = (123, 16, 16, 9, 14, 11)
BOND_DIMS = (12, 27, 7)
BOND_CENTERS = np.arange(0.0, 2.0, 0.1).astype(np.float32)
ANGLE_CENTERS = np.arange(0.0, np.pi, 0.1).astype(np.float32)
DIHED_CENTERS = np.arange(-np.pi, np.pi, 0.2).astype(np.float32)


def _offsets(dims):
    offs = [0]
    for d in dims[:-1]:
        offs.append(offs[-1] + d)
    return tuple(offs)


def _embed_body(x_ref, t_ref, o_ref, *, offs, k_pad):
    x = x_ref[...]
    b = x.shape[0]
    iota = jax.lax.broadcasted_iota(jnp.int32, (b, k_pad), 1)
    oh = jnp.zeros((b, k_pad), jnp.float32)
    for f, off in enumerate(offs):
        oh = oh + (iota == (x[:, f][:, None] + off)).astype(jnp.float32)
    o_ref[...] = jnp.dot(oh, t_ref[...], preferred_element_type=jnp.float32)


def _embed_call(x, tables, dims, block):
    n = x.shape[0]
    offs = _offsets(dims)
    k = sum(dims)
    k_pad = ((k + 7) // 8) * 8
    t = jnp.zeros((k_pad, LATENT), jnp.float32)
    row = 0
    for tab, d in zip(tables, dims):
        t = jax.lax.dynamic_update_slice(t, tab, (row, 0))
        row += d
    grid = n // block
    return pl.pallas_call(
        functools.partial(_embed_body, offs=offs, k_pad=k_pad),
        grid=(grid,),
        in_specs=[
            pl.BlockSpec((block, len(dims)), lambda i: (i, 0)),
            pl.BlockSpec((k_pad, LATENT), lambda i: (0, 0)),
        ],
        out_specs=pl.BlockSpec((block, LATENT), lambda i: (i, 0)),
        out_shape=jax.ShapeDtypeStruct((n, LATENT), jnp.float32),
    )(x, t)


def _bondfeat_body(x_ref, v_ref, t_ref, b_ref, c_ref, o_ref, *, offs, k_pad):
    x = x_ref[...]
    v = v_ref[0, 0, :]
    b = x.shape[0]
    iota = jax.lax.broadcasted_iota(jnp.int32, (b, k_pad), 1)
    feat = jnp.zeros((b, k_pad), jnp.float32)
    for f, off in enumerate(offs):
        feat = feat + (iota == (x[:, f][:, None] + off)).astype(jnp.float32)
    cvec = c_ref[...]
    feat = feat + jnp.exp(-GAMMA * (v[:, None] - cvec[None, :]) ** 2)
    o_ref[...] = (
        jnp.dot(feat, t_ref[...], preferred_element_type=jnp.float32)
        + b_ref[...][None, :]
    )


def _bondfeat_call(bond_attr, bond_lengths, tables, rbf_p, block):
    n = bond_attr.shape[0]
    offs = _offsets(BOND_DIMS)
    k_emb = sum(BOND_DIMS)
    n_c = BOND_CENTERS.shape[0]
    k = k_emb + n_c
    k_pad = ((k + 7) // 8) * 8
    t = jnp.zeros((k_pad, LATENT), jnp.float32)
    row = 0
    for tab, d in zip(tables, BOND_DIMS):
        t = jax.lax.dynamic_update_slice(t, tab, (row, 0))
        row += d
    t = jax.lax.dynamic_update_slice(t, rbf_p["W"], (k_emb, 0))
    c_full = np.full((k_pad,), 1.0e4, np.float32)
    c_full[k_emb:k_emb + n_c] = BOND_CENTERS
    grid = n // block
    return pl.pallas_call(
        functools.partial(_bondfeat_body, offs=offs, k_pad=k_pad),
        grid=(grid,),
        in_specs=[
            pl.BlockSpec((block, 3), lambda i: (i, 0)),
            pl.BlockSpec((1, 1, block), lambda i: (i, 0, 0)),
            pl.BlockSpec((k_pad, LATENT), lambda i: (0, 0)),
            pl.BlockSpec((LATENT,), lambda i: (0,)),
            pl.BlockSpec((k_pad,), lambda i: (0,)),
        ],
        out_specs=pl.BlockSpec((block, LATENT), lambda i: (i, 0)),
        out_shape=jax.ShapeDtypeStruct((n, LATENT), jnp.float32),
    )(bond_attr, bond_lengths.reshape(grid, 1, block), t, rbf_p["b"],
      jnp.asarray(c_full))


def _rbf_body(v_ref, w_ref, b_ref, c_ref, o_ref):
    v = v_ref[0, 0, :]
    cvec = c_ref[...]
    r = jnp.exp(-GAMMA * (v[:, None] - cvec[None, :]) ** 2)
    o_ref[...] = (
        jnp.dot(r, w_ref[...], preferred_element_type=jnp.float32)
        + b_ref[...][None, :]
    )


def _rbf_call(v, centers, p, block):
    n = v.shape[0]
    c = centers.shape[0]
    grid = n // block
    return pl.pallas_call(
        _rbf_body,
        grid=(grid,),
        in_specs=[
            pl.BlockSpec((1, 1, block), lambda i: (i, 0, 0)),
            pl.BlockSpec((c, LATENT), lambda i: (0, 0)),
            pl.BlockSpec((LATENT,), lambda i: (0,)),
            pl.BlockSpec((c,), lambda i: (0,)),
        ],
        out_specs=pl.BlockSpec((block, LATENT), lambda i: (i, 0)),
        out_shape=jax.ShapeDtypeStruct((n, LATENT), jnp.float32),
    )(v.reshape(grid, 1, block), p["W"], p["b"], jnp.asarray(centers))


def _mlp_body(agg_ref, skip_ref, w1_ref, b1_ref, w2_ref, b2_ref, g_ref,
              bl_ref, o_ref, *, last_act):
    agg = agg_ref[...]
    h = jnp.dot(agg, w1_ref[...], preferred_element_type=jnp.float32)
    h = jnp.maximum(h + b1_ref[...][None, :], 0.0)
    o = jnp.dot(h, w2_ref[...], preferred_element_type=jnp.float32)
    o = o + b2_ref[...][None, :]
    mu = jnp.mean(o, axis=-1, keepdims=True)
    var = jnp.mean((o - mu) ** 2, axis=-1, keepdims=True)
    o = (o - mu) * jax.lax.rsqrt(var + 1e-5)
    o = o * g_ref[...][None, :] + bl_ref[...][None, :]
    if last_act:
        o = jnp.maximum(o, 0.0)
    o_ref[...] = o + skip_ref[...]


def _mlp_call(agg, skip, p, last_act, block):
    n = skip.shape[0]
    grid = n // block
    return pl.pallas_call(
        functools.partial(_mlp_body, last_act=last_act),
        grid=(grid,),
        in_specs=[
            pl.BlockSpec((block, LATENT), lambda i: (i, 0)),
            pl.BlockSpec((block, LATENT), lambda i: (i, 0)),
            pl.BlockSpec((LATENT, 2 * LATENT), lambda i: (0, 0)),
            pl.BlockSpec((2 * LATENT,), lambda i: (0,)),
            pl.BlockSpec((2 * LATENT, LATENT), lambda i: (0, 0)),
            pl.BlockSpec((LATENT,), lambda i: (0,)),
            pl.BlockSpec((LATENT,), lambda i: (0,)),
            pl.BlockSpec((LATENT,), lambda i: (0,)),
        ],
        out_specs=pl.BlockSpec((block, LATENT), lambda i: (i, 0)),
        out_shape=jax.ShapeDtypeStruct((n, LATENT), jnp.float32),
    )(agg, skip, p["W1"], p["b1"], p["W2"], p["b2"], p["ln_g"], p["ln_b"])


def _pool_body(batch_ref, node_ref, o_ref, seg_ref, cnt_ref, *, n_graphs,
               n_steps):
    i = pl.program_id(0)

    @pl.when(i == 0)
    def _():
        seg_ref[...] = jnp.zeros_like(seg_ref)
        cnt_ref[...] = jnp.zeros_like(cnt_ref)

    batch = batch_ref[0, 0, :]
    b = batch.shape[0]
    gid = jax.lax.broadcasted_iota(jnp.int32, (n_graphs, b), 0)
    oh = (gid == batch[None, :]).astype(jnp.float32)
    seg_ref[...] += jnp.dot(oh, node_ref[...],
                            preferred_element_type=jnp.float32)
    cnt_ref[...] += jnp.dot(oh, jnp.ones((b, LATENT), jnp.float32),
                            preferred_element_type=jnp.float32)

    @pl.when(i == n_steps - 1)
    def _():
        o_ref[...] = seg_ref[...] / jnp.maximum(cnt_ref[...], 1.0)


def _pool_call(node, batch, n_graphs, block):
    n = node.shape[0]
    grid = n // block
    return pl.pallas_call(
        functools.partial(_pool_body, n_graphs=n_graphs, n_steps=grid),
        grid=(grid,),
        in_specs=[
            pl.BlockSpec((1, 1, block), lambda i: (i, 0, 0)),
            pl.BlockSpec((block, LATENT), lambda i: (i, 0)),
        ],
        out_specs=pl.BlockSpec((n_graphs, LATENT), lambda i: (0, 0)),
        out_shape=jax.ShapeDtypeStruct((n_graphs, LATENT), jnp.float32),
        scratch_shapes=[
            pltpu.VMEM((n_graphs, LATENT), jnp.float32),
            pltpu.VMEM((n_graphs, LATENT), jnp.float32),
        ],
    )(batch.reshape(grid, 1, block), node)


_SC_R = 4096
_SC_LOGR = 12
_SC_G = 128
_SC_PAD = 16 * _SC_G + 8


def _sc_agg_body(node_hbm, edge_hbm, idx4_hbm, co_hbm, out_hbm,
                 idx0, idx1, ra0, ra1, rb0, rb1, dl2d, zbuf, cob, acc,
                 si0, si1, sa0, sa1, sb0, sb1, ss0, ss1,
                 *, n_chunks, ng_tot):
    cid = lax.axis_index("c")
    sid = lax.axis_index("s")
    r = _SC_R
    g = _SC_G
    npairs = n_chunks // 2
    idxb = (idx0, idx1)
    ra = (ra0, ra1)
    rb = (rb0, rb1)
    si = (si0, si1)
    sa = (sa0, sa1)
    sb = (sb0, sb1)
    ss = (ss0, ss1)
    pltpu.sync_copy(co_hbm, cob)

    def zero_z(i, _):
        for q in range(8):
            zbuf[i, pl.ds(16 * q, 16)] = jnp.zeros((16,), jnp.float32)
        return _

    lax.fori_loop(0, g, zero_z, None)
    row0 = sid * (r // 16)

    def fire_idx(gidx, b):
        pltpu.async_copy(idx4_hbm.at[jnp.minimum(gidx, ng_tot - 1)],
                         idxb[b], si[b])

    def wait_idx(b):
        pltpu.make_async_copy(idx4_hbm.at[0], idxb[b], si[b]).wait()

    def fire_gathers(b):
        pltpu.async_copy(node_hbm.at[idxb[b].at[0]], ra[b], sa[b])
        pltpu.async_copy(edge_hbm.at[idxb[b].at[1]], rb[b], sb[b])

    def wait_gathers(b):
        pltpu.make_async_copy(node_hbm.at[idxb[b].at[0]], ra[b], sa[b]).wait()
        pltpu.make_async_copy(edge_hbm.at[idxb[b].at[1]], rb[b], sb[b]).wait()

    def fire_scatter(b):
        pltpu.async_copy(ra[b], acc.at[dl2d.at[b]], ss[b], add=True)

    def wait_scatter(b):
        pltpu.make_async_copy(ra[b], acc.at[dl2d.at[b]], ss[b]).wait()

    def chunk_body(k, _):
        ci = 2 * k + cid
        lo = ci * r
        ent = pl.multiple_of((cid * npairs + k) * 8, 8)
        cov = cob[pl.ds(ent, 16)]
        gstart = cov[0] >> 7
        gend = (cov[1] + g - 1) >> 7
        ng_u = (gend - gstart + 15) >> 4
        n_pairs_g = (ng_u + 1) >> 1

        def valid(kk):
            return (gstart + kk * 16 + sid) * _SC_G < cov[1]

        def gidx_of(kk):
            return gstart + kk * 16 + sid

        def compute(b):
            def relu_row(rr, _):
                for q in range(8):
                    sl = pl.ds(16 * q, 16)
                    ra[b][rr, sl] = jnp.maximum(
                        ra[b][rr, sl] + rb[b][rr, sl], 0.0)
                return _

            lax.fori_loop(0, _SC_G, relu_row, None, unroll=4)
            for q in range(8):
                sl = pl.ds(16 * q, 16)
                dl2d[b, sl] = jnp.where(idxb[b][2, sl] == ci,
                                        idxb[b][3, sl], _SC_R)

        def pair_body(k2, _):
            for nb in range(2):
                kk = 2 * k2 + nb
                other = 1 - nb

                @pl.when(valid(kk))
                def _wg():
                    wait_gathers(nb)

                @pl.when((k2 + nb > 0) & valid(kk - 1))
                def _ws():
                    wait_scatter(other)

                wait_idx(other)

                @pl.when(valid(kk + 1))
                def _fg():
                    fire_gathers(other)

                @pl.when(valid(kk))
                def _cs():
                    compute(nb)
                    fire_scatter(nb)

                fire_idx(gidx_of(kk + 2), nb)
            return _

        fire_idx(gidx_of(0), 0)
        fire_idx(gidx_of(1), 1)
        wait_idx(0)

        @pl.when(valid(0))
        def _fg0():
            fire_gathers(0)

        lax.fori_loop(0, n_pairs_g, pair_body, None)
        wait_idx(1)

        @pl.when(valid(2 * n_pairs_g))
        def _dg():
            wait_gathers(0)

        @pl.when((n_pairs_g > 0) & valid(2 * n_pairs_g - 1))
        def _dsd():
            wait_scatter(1)
        plsc.subcore_barrier()

        out_base = lo + row0
        for o in range(0, r // 16, g):
            pltpu.sync_copy(acc.at[pl.ds(row0 + o, g)],
                            out_hbm.at[pl.ds(out_base + o, g)])
            pltpu.sync_copy(zbuf, acc.at[pl.ds(row0 + o, g)])
        plsc.subcore_barrier()
        return _

    for o in range(0, r // 16, g):
        pltpu.sync_copy(zbuf, acc.at[pl.ds(row0 + o, g)])
    plsc.subcore_barrier()
    lax.fori_loop(0, npairs, chunk_body, None)


def _sc_prep(edges, n_nodes):
    src = edges[0].astype(jnp.int32)
    dst = edges[1].astype(jnp.int32)
    e = src.shape[0]
    n_chunks = -(-n_nodes // _SC_R)
    if n_chunks % 2:
        n_chunks += 1
    perm = jnp.argsort(dst).astype(jnp.int32)
    dst_s = jnp.take(dst, perm)
    src_s = jnp.take(src, perm)
    dl_s = dst_s & (_SC_R - 1)
    ck_s = dst_s >> _SC_LOGR
    co = jnp.searchsorted(
        dst_s, jnp.arange(n_chunks + 1, dtype=jnp.int32) * _SC_R
    ).astype(jnp.int32)
    npairs = n_chunks // 2
    ci_all = jnp.concatenate([
        jnp.arange(0, n_chunks, 2, dtype=jnp.int32),
        jnp.arange(1, n_chunks, 2, dtype=jnp.int32),
    ])
    pairs = jnp.stack([co[ci_all], co[ci_all + 1]], axis=1)
    cob = jnp.zeros((2 * npairs, 8), jnp.int32)
    cob = jax.lax.dynamic_update_slice(cob, pairs, (0, 0))
    cob = jnp.concatenate([cob.reshape(-1), jnp.zeros((16,), jnp.int32)])
    ng_tot = -(-e // _SC_G) + 17
    pad = ng_tot * _SC_G - e
    zpad = jnp.zeros((pad,), jnp.int32)
    idx4 = jnp.stack([
        jnp.concatenate([src_s, zpad]).reshape(ng_tot, _SC_G),
        jnp.concatenate([perm, zpad]).reshape(ng_tot, _SC_G),
        jnp.concatenate([ck_s, zpad + (1 << 30)]).reshape(ng_tot, _SC_G),
        jnp.concatenate([dl_s, zpad]).reshape(ng_tot, _SC_G),
    ], axis=1)
    return dict(n_chunks=n_chunks, ng_tot=ng_tot, idx4=idx4, co=cob)


def _sc_message_agg(node_hidden, edge_hidden, prep, n_nodes):
    n_chunks = prep["n_chunks"]
    n_pad = n_chunks * _SC_R
    kfn = functools.partial(
        pl.kernel,
        mesh=plsc.VectorSubcoreMesh(core_axis_name="c", subcore_axis_name="s"),
        out_type=jax.ShapeDtypeStruct((n_pad, LATENT), jnp.float32),
        scratch_types=[
            pltpu.VMEM((4, _SC_G), jnp.int32),
            pltpu.VMEM((4, _SC_G), jnp.int32),
            pltpu.VMEM((_SC_G, LATENT), jnp.float32),
            pltpu.VMEM((_SC_G, LATENT), jnp.float32),
            pltpu.VMEM((_SC_G, LATENT), jnp.float32),
            pltpu.VMEM((_SC_G, LATENT), jnp.float32),
            pltpu.VMEM((2, 128), jnp.int32),
            pltpu.VMEM((_SC_G, LATENT), jnp.float32),
            pltpu.VMEM((2 * (prep["n_chunks"] // 2) * 8 + 16,),
                       jnp.int32),
            pltpu.VMEM_SHARED((_SC_R + 8, LATENT), jnp.float32),
            pltpu.SemaphoreType.DMA,
            pltpu.SemaphoreType.DMA,
            pltpu.SemaphoreType.DMA,
            pltpu.SemaphoreType.DMA,
            pltpu.SemaphoreType.DMA,
            pltpu.SemaphoreType.DMA,
            pltpu.SemaphoreType.DMA,
            pltpu.SemaphoreType.DMA,
        ],
    )(functools.partial(_sc_agg_body, n_chunks=n_chunks,
                        ng_tot=prep["ng_tot"]))
    return kfn(node_hidden, edge_hidden, prep["idx4"], prep["co"])


def _message_agg(node_hidden, edge_hidden, prep, n_nodes):
    return _sc_message_agg(node_hidden, edge_hidden, prep, n_nodes)


def _block(node_hidden, edge_hidden, prep, p, last_act, block):
    agg = _message_agg(node_hidden, edge_hidden, prep,
                       node_hidden.shape[0])
    return _mlp_call(agg, node_hidden, p, last_act, block)


def kernel(AtomBondGraph_edges, BondAngleGraph_edges, AngleDihedralGraph_edges,
           x, bond_attr, bond_lengths, bond_angles, dihedral_angles,
           atom_batch, num_graphs, masked_atom_indices, masked_bond_indices,
           masked_angle_indices, masked_dihedral_indices, params):
    n_atoms = x.shape[0]
    n_bonds = bond_attr.shape[0]

    for i, d in enumerate(ATOM_DIMS):
        x = x.at[masked_atom_indices, i].set(d - 1)
    for i, d in enumerate(BOND_DIMS):
        bond_attr = bond_attr.at[masked_bond_indices, i].set(d - 1)
    bond_lengths = bond_lengths.at[masked_bond_indices].set(0.0)
    bond_angles = bond_angles.at[masked_angle_indices].set(0.0)
    dihedral_angles = dihedral_angles.at[masked_dihedral_indices].set(0.0)

    ab = 2000
    eb = 4000

    node_hidden = _embed_call(x, params["init_atom_emb"], ATOM_DIMS, ab)
    bond_hidden = _bondfeat_call(bond_attr, bond_lengths,
                                 params["init_bond_emb"],
                                 params["init_bond_rbf"], eb)
    angle_hidden = _rbf_call(bond_angles, ANGLE_CENTERS,
                             params["init_angle_rbf"], eb)

    ab_prep = _sc_prep(AtomBondGraph_edges, n_atoms)
    ba_prep = _sc_prep(BondAngleGraph_edges, n_bonds)
    ad_prep = _sc_prep(AngleDihedralGraph_edges, bond_angles.shape[0])

    cur_dihedral_hidden = None
    n_layers = len(params["layers"])
    for l in range(n_layers):
        lp = params["layers"][l]
        last_act = (l != n_layers - 1)
        new_node = _block(node_hidden, bond_hidden, ab_prep,
                          lp["ab_block"], last_act, ab)
        cur_edge_hidden = _bondfeat_call(bond_attr, bond_lengths,
                                         lp["bond_emb"], lp["bond_rbf"], eb)
        new_bond = _block(cur_edge_hidden, angle_hidden, ba_prep,
                          lp["ba_block"], last_act, eb)
        cur_angle_hidden = _rbf_call(bond_angles, ANGLE_CENTERS,
                                     lp["angle_rbf"], eb)
        cur_dihedral_hidden = _rbf_call(dihedral_angles, DIHED_CENTERS,
                                        lp["dihed_rbf"], eb)
        new_angle = _block(cur_angle_hidden, cur_dihedral_hidden,
                           ad_prep, lp["ad_block"],
                           last_act, eb)
        node_hidden, bond_hidden, angle_hidden = new_node, new_bond, new_angle

    graph_repr = _pool_call(node_hidden, atom_batch, 128, 1000)
    return (node_hidden, bond_hidden, angle_hidden, cur_dihedral_hidden,
            graph_repr)
